# baseline (speedup 1.0000x reference)
import functools
import math

import jax
import jax.numpy as jnp
from jax import lax
from jax.experimental import pallas as pl
from jax.experimental.pallas import tpu as pltpu
from jax.experimental.pallas import tpu_sc as plsc

D_MODEL = 768
SEQ_LEN = 4096
MAX_LEN = 8192
BATCH = 4
LANES = 16
KVECS = D_MODEL // LANES

NUM_WORKERS = 32
S_PER_W = SEQ_LEN // NUM_WORKERS
ROWS_PER_W = S_PER_W * BATCH
S_CHUNK = 8
ROWS_CHUNK = S_CHUNK * BATCH
CHUNKS = S_PER_W // S_CHUNK
NOUT = 3
SCALE = math.sqrt(D_MODEL)


def _emb_pe_kernel(x_hbm, pe_hbm, table_hbm, out_hbm,
                   idx_all, emb0, emb1, out0, out1, out2, pe_v,
                   sem_g, sem_pe, sem_out):
    wid = lax.axis_index("s") * 2 + lax.axis_index("c")
    row0 = wid * ROWS_PER_W
    s0 = wid * S_PER_W
    embs = (emb0, emb1)
    outs = (out0, out1, out2)

    pltpu.sync_copy(x_hbm.at[pl.ds(row0, ROWS_PER_W)], idx_all)

    def gather_desc(c, j):
        idx_slice = idx_all.at[pl.ds(c * ROWS_CHUNK, ROWS_CHUNK)]
        return pltpu.make_async_copy(table_hbm.at[idx_slice], embs[j],
                                     sem_g.at[j])

    def pe_desc(c):
        src = pe_hbm.at[pl.ds(s0 + c * S_CHUNK, S_CHUNK), 0, :]
        return pltpu.make_async_copy(src, pe_v, sem_pe)

    def out_desc(c, jo):
        dst = out_hbm.at[pl.ds(s0 + c * S_CHUNK, S_CHUNK)]
        return pltpu.make_async_copy(outs[jo], dst, sem_out.at[jo])

    def compute(j, jo):
        emb_v, out_v = embs[j], outs[jo]

        def s_body(sl, carry):
            @plsc.parallel_loop(0, KVECS, unroll=4)
            def k_body(kk):
                off = kk * LANES
                pev = pe_v[sl, pl.ds(off, LANES)]
                for b in range(BATCH):
                    out_v[sl, b, pl.ds(off, LANES)] = (
                        emb_v[sl * BATCH + b, pl.ds(off, LANES)] * SCALE + pev
                    )

            return carry

        lax.fori_loop(0, S_CHUNK, s_body, 0)

    gather_desc(0, 0).start()
    gather_desc(1, 1).start()
    pe_desc(0).start()
    for c in range(CHUNKS):
        j = c % 2
        jo = c % NOUT
        gather_desc(c, j).wait()
        pe_desc(c).wait()
        if c >= NOUT:
            out_desc(c - NOUT, jo).wait()
        compute(j, jo)
        if c + 1 < CHUNKS:
            pe_desc(c + 1).start()
        out_desc(c, jo).start()
        if c + 2 < CHUNKS:
            gather_desc(c + 2, j).start()
    for c in range(CHUNKS - NOUT, CHUNKS):
        out_desc(c, c % NOUT).wait()


def kernel(x, table, pe):
    xf = x.reshape(SEQ_LEN * BATCH)
    mesh = plsc.VectorSubcoreMesh(core_axis_name="c", subcore_axis_name="s")
    run = functools.partial(
        pl.kernel,
        mesh=mesh,
        out_type=jax.ShapeDtypeStruct((SEQ_LEN, BATCH, D_MODEL), jnp.float32),
        scratch_types=[
            pltpu.VMEM((ROWS_PER_W,), jnp.int32),
            pltpu.VMEM((ROWS_CHUNK, D_MODEL), jnp.float32),
            pltpu.VMEM((ROWS_CHUNK, D_MODEL), jnp.float32),
            pltpu.VMEM((S_CHUNK, BATCH, D_MODEL), jnp.float32),
            pltpu.VMEM((S_CHUNK, BATCH, D_MODEL), jnp.float32),
            pltpu.VMEM((S_CHUNK, BATCH, D_MODEL), jnp.float32),
            pltpu.VMEM((S_CHUNK, D_MODEL), jnp.float32),
            pltpu.SemaphoreType.DMA((2,)),
            pltpu.SemaphoreType.DMA,
            pltpu.SemaphoreType.DMA((NOUT,)),
        ],
    )(_emb_pe_kernel)
    return run(xf, pe, table)


# speedup vs baseline: 1.1367x; 1.1367x over previous
import functools
import math

import jax
import jax.numpy as jnp
from jax import lax
from jax.experimental import pallas as pl
from jax.experimental.pallas import tpu as pltpu
from jax.experimental.pallas import tpu_sc as plsc

D_MODEL = 768
SEQ_LEN = 4096
MAX_LEN = 8192
BATCH = 4
LANES = 16
KVECS = D_MODEL // LANES

NUM_WORKERS = 32
S_PER_W = SEQ_LEN // NUM_WORKERS
ROWS_PER_W = S_PER_W * BATCH
S_CHUNK = 8
ROWS_CHUNK = S_CHUNK * BATCH
CHUNKS = S_PER_W // S_CHUNK
SCALE = math.sqrt(D_MODEL)


def _emb_pe_kernel(x_hbm, pe_hbm, table_hbm, out_hbm,
                   idx_all, emb0, emb1, out0, out1, pe0, pe1,
                   sem_g, sem_pe, sem_out):
    wid = lax.axis_index("s") * 2 + lax.axis_index("c")
    row0 = wid * ROWS_PER_W
    s0 = wid * S_PER_W
    embs = (emb0, emb1)
    outs = (out0, out1)
    pes = (pe0, pe1)

    pltpu.sync_copy(x_hbm.at[pl.ds(row0, ROWS_PER_W)], idx_all)

    def gather_desc(c, j):
        idx_slice = idx_all.at[pl.ds(c * ROWS_CHUNK, ROWS_CHUNK)]
        return pltpu.make_async_copy(table_hbm.at[idx_slice], embs[j],
                                     sem_g.at[j])

    def pe_desc(c, j):
        src = pe_hbm.at[pl.ds(s0 + c * S_CHUNK, S_CHUNK), 0, :]
        return pltpu.make_async_copy(src, pes[j], sem_pe.at[j])

    def out_desc(c, j):
        dst = out_hbm.at[pl.ds(s0 + c * S_CHUNK, S_CHUNK)]
        return pltpu.make_async_copy(outs[j], dst, sem_out.at[j])

    def compute(j):
        emb_v, out_v, pe_v = embs[j], outs[j], pes[j]

        def s_body(sl, carry):
            @plsc.parallel_loop(0, KVECS, unroll=8)
            def k_body(kk):
                off = kk * LANES
                pev = pe_v[sl, pl.ds(off, LANES)]
                for b in range(BATCH):
                    out_v[sl, b, pl.ds(off, LANES)] = (
                        emb_v[sl * BATCH + b, pl.ds(off, LANES)] * SCALE + pev
                    )

            return carry

        lax.fori_loop(0, S_CHUNK, s_body, 0)

    gather_desc(0, 0).start()
    pe_desc(0, 0).start()
    gather_desc(1, 1).start()
    pe_desc(1, 1).start()
    for c in range(CHUNKS):
        j = c % 2
        gather_desc(c, j).wait()
        pe_desc(c, j).wait()
        if c >= 2:
            out_desc(c - 2, j).wait()
        compute(j)
        out_desc(c, j).start()
        if c + 2 < CHUNKS:
            gather_desc(c + 2, j).start()
            pe_desc(c + 2, j).start()
    out_desc(CHUNKS - 2, 0).wait()
    out_desc(CHUNKS - 1, 1).wait()


def kernel(x, table, pe):
    xf = x.reshape(SEQ_LEN * BATCH)
    mesh = plsc.VectorSubcoreMesh(core_axis_name="c", subcore_axis_name="s")
    run = functools.partial(
        pl.kernel,
        mesh=mesh,
        out_type=jax.ShapeDtypeStruct((SEQ_LEN, BATCH, D_MODEL), jnp.float32),
        scratch_types=[
            pltpu.VMEM((ROWS_PER_W,), jnp.int32),
            pltpu.VMEM((ROWS_CHUNK, D_MODEL), jnp.float32),
            pltpu.VMEM((ROWS_CHUNK, D_MODEL), jnp.float32),
            pltpu.VMEM((S_CHUNK, BATCH, D_MODEL), jnp.float32),
            pltpu.VMEM((S_CHUNK, BATCH, D_MODEL), jnp.float32),
            pltpu.VMEM((S_CHUNK, D_MODEL), jnp.float32),
            pltpu.VMEM((S_CHUNK, D_MODEL), jnp.float32),
            pltpu.SemaphoreType.DMA((2,)),
            pltpu.SemaphoreType.DMA((2,)),
            pltpu.SemaphoreType.DMA((2,)),
        ],
    )(_emb_pe_kernel)
    return run(xf, pe, table)
